# baseline (speedup 1.0000x reference)
import jax
import jax.numpy as jnp
from jax import lax
from jax.experimental import pallas as pl
from jax.experimental.pallas import tpu as pltpu
from jax.experimental.pallas import tpu_sc as plsc

VOCAB = 1000000
EMBED = 32
BATCH = 16384
HIST = 200

NC, NS = 2, 16
NW = NC * NS
ROWS_PER_W = BATCH // NW
G = 8
NCHUNK = ROWS_PER_W // G
NPAIR = NCHUNK // 2
IDX_PER_CHUNK = G * HIST
NGATHER = 32
IDX_PER_GATHER = IDX_PER_CHUNK // NGATHER
UNROLL = 8
NTRIP = HIST // UNROLL
OUT_ROWS = BATCH * 2 // 16


def _sc_kernel(x2_hbm, table_hbm, params_hbm, out_hbm,
               idx0_v, idx1_v, rows0_v, rows1_v, out_v, par_v, sem0, sem1):
    cid = lax.axis_index("c")
    sid = lax.axis_index("s")
    wid = sid * NC + cid

    pltpu.sync_copy(params_hbm, par_v)
    w00 = par_v[0, :]
    w01 = par_v[1, :]
    w10 = par_v[2, :]
    w11 = par_v[3, :]
    bvec = par_v[4, :]
    lane = lax.iota(jnp.int32, 16)
    rot_idx = [lax.rem(lane + k, 16).reshape(16, 1) for k in (8, 4, 2, 1)]
    dnums = lax.GatherDimensionNumbers(
        offset_dims=(), collapsed_slice_dims=(0,), start_index_map=(0,))

    def lanesum(v):
        for idx in rot_idx:
            v = v + lax.gather(
                v, idx, dimension_numbers=dnums, slice_sizes=(1,),
                mode=lax.GatherScatterMode.PROMISE_IN_BOUNDS)
        return v

    def stage(g, idx_v, rows_v, sem):
        row0 = (wid * NCHUNK + g) * NGATHER
        pltpu.sync_copy(x2_hbm.at[pl.ds(row0, NGATHER), :], idx_v)
        for j in range(NGATHER):
            pltpu.async_copy(
                table_hbm.at[idx_v.at[j]],
                rows_v.at[pl.ds(j * IDX_PER_GATHER, IDX_PER_GATHER), :],
                sem,
            )

    def drain(idx_v, rows_v, sem):
        for j in range(NGATHER):
            pltpu.make_async_copy(
                table_hbm.at[idx_v.at[j]],
                rows_v.at[pl.ds(j * IDX_PER_GATHER, IDX_PER_GATHER), :],
                sem,
            ).wait()

    def compute(g, rows_v):
        def inner(l, accs):
            accs = list(accs)
            i0 = UNROLL * l
            for r in range(G):
                base = r * HIST + i0
                a, b = accs[2 * r], accs[2 * r + 1]
                for k in range(UNROLL):
                    a = a + rows_v[base + k, pl.ds(0, 16)]
                    b = b + rows_v[base + k, pl.ds(16, 16)]
                accs[2 * r], accs[2 * r + 1] = a, b
            return tuple(accs)

        z = jnp.zeros((16,), jnp.float32)
        accs = lax.fori_loop(0, NTRIP, inner, (z,) * (2 * G))

        outvec = jnp.zeros((16,), jnp.float32)
        for r in range(G):
            s0, s1 = accs[2 * r], accs[2 * r + 1]
            o0 = lanesum(s0 * w00 + s1 * w01)
            o1 = lanesum(s0 * w10 + s1 * w11)
            outvec = jnp.where(lane == 2 * r, o0, outvec)
            outvec = jnp.where(lane == 2 * r + 1, o1, outvec)

        out_v[g, :] = outvec + bvec

    stage(0, idx0_v, rows0_v, sem0)

    def pair_body(p, carry):
        c0 = 2 * p
        stage(c0 + 1, idx1_v, rows1_v, sem1)
        drain(idx0_v, rows0_v, sem0)
        compute(c0, rows0_v)
        c2 = jnp.minimum(c0 + 2, NCHUNK - 1)
        stage(c2, idx0_v, rows0_v, sem0)
        drain(idx1_v, rows1_v, sem1)
        compute(c0 + 1, rows1_v)
        return carry

    lax.fori_loop(0, NPAIR, pair_body, 0)
    drain(idx0_v, rows0_v, sem0)
    pltpu.sync_copy(out_v, out_hbm.at[pl.ds(wid * NCHUNK, NCHUNK), :])


@jax.jit
def kernel(x, table, fc_w, fc_b):
    x2 = x.reshape(BATCH * HIST // IDX_PER_GATHER, IDX_PER_GATHER)
    w = (fc_w.astype(jnp.float32) / HIST).reshape(4, 16)
    bvec = jnp.tile(fc_b.astype(jnp.float32), 8).reshape(1, 16)
    params = jnp.concatenate([w, bvec], axis=0)

    mesh = plsc.VectorSubcoreMesh(core_axis_name="c", subcore_axis_name="s")
    run = pl.kernel(
        _sc_kernel,
        mesh=mesh,
        out_type=jax.ShapeDtypeStruct((OUT_ROWS, 16), jnp.float32),
        scratch_types=[
            pltpu.VMEM((NGATHER, IDX_PER_GATHER), jnp.int32),
            pltpu.VMEM((NGATHER, IDX_PER_GATHER), jnp.int32),
            pltpu.VMEM((IDX_PER_CHUNK, EMBED), jnp.float32),
            pltpu.VMEM((IDX_PER_CHUNK, EMBED), jnp.float32),
            pltpu.VMEM((NCHUNK, 16), jnp.float32),
            pltpu.VMEM((5, 16), jnp.float32),
            pltpu.SemaphoreType.DMA,
            pltpu.SemaphoreType.DMA,
        ],
        compiler_params=pltpu.CompilerParams(use_tc_tiling_on_sc=False),
    )
    out16 = run(x2, table, params)
    return out16.reshape(BATCH, 2)


# speedup vs baseline: 1.0507x; 1.0507x over previous
import jax
import jax.numpy as jnp
from jax import lax
from jax.experimental import pallas as pl
from jax.experimental.pallas import tpu as pltpu
from jax.experimental.pallas import tpu_sc as plsc

VOCAB = 1000000
EMBED = 32
BATCH = 16384
HIST = 200

NC, NS = 2, 16
NW = NC * NS
ROWS_PER_W = BATCH // NW
G = 8
NCHUNK = ROWS_PER_W // G
NPAIR = NCHUNK // 2
IDX_PER_CHUNK = G * HIST
NGATHER = 16
IDX_PER_GATHER = IDX_PER_CHUNK // NGATHER
OUT_ROWS = BATCH * 2 // 16


def _sc_kernel(x2_hbm, table_hbm, params_hbm, out_hbm,
               idx0_v, idx1_v, rows0_v, rows1_v, out_v, par_v, sem0, sem1):
    cid = lax.axis_index("c")
    sid = lax.axis_index("s")
    wid = sid * NC + cid

    pltpu.sync_copy(params_hbm, par_v)
    w00 = par_v[0, :]
    w01 = par_v[1, :]
    w10 = par_v[2, :]
    w11 = par_v[3, :]
    bvec = par_v[4, :]
    lane = lax.iota(jnp.int32, 16)
    rot_idx = [lax.rem(lane + k, 16).reshape(16, 1) for k in (8, 4, 2, 1)]
    dnums = lax.GatherDimensionNumbers(
        offset_dims=(), collapsed_slice_dims=(0,), start_index_map=(0,))

    def lanesum(v):
        for idx in rot_idx:
            v = v + lax.gather(
                v, idx, dimension_numbers=dnums, slice_sizes=(1,),
                mode=lax.GatherScatterMode.PROMISE_IN_BOUNDS)
        return v

    def stage(g, idx_v, rows_v, sem):
        row0 = (wid * NCHUNK + g) * NGATHER
        pltpu.sync_copy(x2_hbm.at[pl.ds(row0, NGATHER), :], idx_v)
        for j in range(NGATHER):
            pltpu.async_copy(
                table_hbm.at[idx_v.at[j]],
                rows_v.at[pl.ds(j * IDX_PER_GATHER, IDX_PER_GATHER), :],
                sem,
            )

    def drain(idx_v, rows_v, sem):
        for j in range(NGATHER):
            pltpu.make_async_copy(
                table_hbm.at[idx_v.at[j]],
                rows_v.at[pl.ds(j * IDX_PER_GATHER, IDX_PER_GATHER), :],
                sem,
            ).wait()

    def compute(g, rows_v):
        outvec = jnp.zeros((16,), jnp.float32)
        for r in range(G):
            base = r * HIST

            def inner(l, accs):
                a0, b0, a1, b1 = accs
                i0 = base + 8 * l
                for k in range(0, 8, 2):
                    a0 = a0 + rows_v[i0 + k, pl.ds(0, 16)]
                    a1 = a1 + rows_v[i0 + k, pl.ds(16, 16)]
                    b0 = b0 + rows_v[i0 + k + 1, pl.ds(0, 16)]
                    b1 = b1 + rows_v[i0 + k + 1, pl.ds(16, 16)]
                return a0, b0, a1, b1

            z = jnp.zeros((16,), jnp.float32)
            a0, b0, a1, b1 = lax.fori_loop(0, HIST // 8, inner, (z, z, z, z))
            s0 = a0 + b0
            s1 = a1 + b1
            o0 = lanesum(s0 * w00 + s1 * w01)
            o1 = lanesum(s0 * w10 + s1 * w11)
            outvec = jnp.where(lane == 2 * r, o0, outvec)
            outvec = jnp.where(lane == 2 * r + 1, o1, outvec)

        out_v[g, :] = outvec + bvec

    stage(0, idx0_v, rows0_v, sem0)

    def pair_body(p, carry):
        c0 = 2 * p
        stage(c0 + 1, idx1_v, rows1_v, sem1)
        drain(idx0_v, rows0_v, sem0)
        compute(c0, rows0_v)

        @pl.when(p < NPAIR - 1)
        def _():
            stage(c0 + 2, idx0_v, rows0_v, sem0)

        drain(idx1_v, rows1_v, sem1)
        compute(c0 + 1, rows1_v)
        return carry

    lax.fori_loop(0, NPAIR, pair_body, 0)
    pltpu.sync_copy(out_v, out_hbm.at[pl.ds(wid * NCHUNK, NCHUNK), :])


@jax.jit
def kernel(x, table, fc_w, fc_b):
    x2 = x.reshape(BATCH * HIST // IDX_PER_GATHER, IDX_PER_GATHER)
    w = (fc_w.astype(jnp.float32) / HIST).reshape(4, 16)
    bvec = jnp.tile(fc_b.astype(jnp.float32), 8).reshape(1, 16)
    params = jnp.concatenate([w, bvec], axis=0)

    mesh = plsc.VectorSubcoreMesh(core_axis_name="c", subcore_axis_name="s")
    run = pl.kernel(
        _sc_kernel,
        mesh=mesh,
        out_type=jax.ShapeDtypeStruct((OUT_ROWS, 16), jnp.float32),
        scratch_types=[
            pltpu.VMEM((NGATHER, IDX_PER_GATHER), jnp.int32),
            pltpu.VMEM((NGATHER, IDX_PER_GATHER), jnp.int32),
            pltpu.VMEM((IDX_PER_CHUNK, EMBED), jnp.float32),
            pltpu.VMEM((IDX_PER_CHUNK, EMBED), jnp.float32),
            pltpu.VMEM((NCHUNK, 16), jnp.float32),
            pltpu.VMEM((5, 16), jnp.float32),
            pltpu.SemaphoreType.DMA,
            pltpu.SemaphoreType.DMA,
        ],
        compiler_params=pltpu.CompilerParams(use_tc_tiling_on_sc=False),
    )
    out16 = run(x2, table, params)
    return out16.reshape(BATCH, 2)


# speedup vs baseline: 1.0673x; 1.0158x over previous
import jax
import jax.numpy as jnp
from jax import lax
from jax.experimental import pallas as pl
from jax.experimental.pallas import tpu as pltpu
from jax.experimental.pallas import tpu_sc as plsc

VOCAB = 1000000
EMBED = 32
BATCH = 16384
HIST = 200

NC, NS = 2, 16
NW = NC * NS
ROWS_PER_W = BATCH // NW
G = 8
NCHUNK = ROWS_PER_W // G
NPAIR = NCHUNK // 2
IDX_PER_CHUNK = G * HIST
SPLITS = [(j * 128, 128) for j in range(12)] + [(1536, 64)]
OUT_ROWS = BATCH * 2 // 16


def _sc_kernel(x2_hbm, table_hbm, params_hbm, out_hbm,
               idx0_v, idx1_v, rows0_v, rows1_v, out_v, par_v, sem0, sem1):
    cid = lax.axis_index("c")
    sid = lax.axis_index("s")
    wid = sid * NC + cid

    pltpu.sync_copy(params_hbm, par_v)
    w00 = par_v[0, :]
    w01 = par_v[1, :]
    w10 = par_v[2, :]
    w11 = par_v[3, :]
    bvec = par_v[4, :]
    lane = lax.iota(jnp.int32, 16)
    rot_idx = [lax.rem(lane + k, 16).reshape(16, 1) for k in (8, 4, 2, 1)]
    dnums = lax.GatherDimensionNumbers(
        offset_dims=(), collapsed_slice_dims=(0,), start_index_map=(0,))

    def lanesum(v):
        for idx in rot_idx:
            v = v + lax.gather(
                v, idx, dimension_numbers=dnums, slice_sizes=(1,),
                mode=lax.GatherScatterMode.PROMISE_IN_BOUNDS)
        return v

    def stage(g, idx_v, rows_v, sem):
        off = (wid * NCHUNK + g) * IDX_PER_CHUNK
        pltpu.sync_copy(x2_hbm.at[pl.ds(off, IDX_PER_CHUNK)], idx_v)
        for start, n in SPLITS:
            pltpu.async_copy(
                table_hbm.at[idx_v.at[pl.ds(start, n)]],
                rows_v.at[pl.ds(start, n), :],
                sem,
            )

    def drain(idx_v, rows_v, sem):
        for start, n in SPLITS:
            pltpu.make_async_copy(
                table_hbm.at[idx_v.at[pl.ds(start, n)]],
                rows_v.at[pl.ds(start, n), :],
                sem,
            ).wait()

    def compute(g, rows_v):
        outvec = jnp.zeros((16,), jnp.float32)
        for r in range(G):
            base = r * HIST

            def inner(l, accs):
                a0, b0, a1, b1 = accs
                i0 = base + 8 * l
                for k in range(0, 8, 2):
                    a0 = a0 + rows_v[i0 + k, pl.ds(0, 16)]
                    a1 = a1 + rows_v[i0 + k, pl.ds(16, 16)]
                    b0 = b0 + rows_v[i0 + k + 1, pl.ds(0, 16)]
                    b1 = b1 + rows_v[i0 + k + 1, pl.ds(16, 16)]
                return a0, b0, a1, b1

            z = jnp.zeros((16,), jnp.float32)
            a0, b0, a1, b1 = lax.fori_loop(0, HIST // 8, inner, (z, z, z, z))
            s0 = a0 + b0
            s1 = a1 + b1
            o0 = lanesum(s0 * w00 + s1 * w01)
            o1 = lanesum(s0 * w10 + s1 * w11)
            outvec = jnp.where(lane == 2 * r, o0, outvec)
            outvec = jnp.where(lane == 2 * r + 1, o1, outvec)

        out_v[g, :] = outvec + bvec

    stage(0, idx0_v, rows0_v, sem0)

    def pair_body(p, carry):
        c0 = 2 * p
        stage(c0 + 1, idx1_v, rows1_v, sem1)
        drain(idx0_v, rows0_v, sem0)
        compute(c0, rows0_v)

        @pl.when(p < NPAIR - 1)
        def _():
            stage(c0 + 2, idx0_v, rows0_v, sem0)

        drain(idx1_v, rows1_v, sem1)
        compute(c0 + 1, rows1_v)
        return carry

    lax.fori_loop(0, NPAIR, pair_body, 0)
    pltpu.sync_copy(out_v, out_hbm.at[pl.ds(wid * NCHUNK, NCHUNK), :])


@jax.jit
def kernel(x, table, fc_w, fc_b):
    x2 = x.reshape(BATCH * HIST)
    w = (fc_w.astype(jnp.float32) / HIST).reshape(4, 16)
    bvec = jnp.tile(fc_b.astype(jnp.float32), 8).reshape(1, 16)
    params = jnp.concatenate([w, bvec], axis=0)

    mesh = plsc.VectorSubcoreMesh(core_axis_name="c", subcore_axis_name="s")
    run = pl.kernel(
        _sc_kernel,
        mesh=mesh,
        out_type=jax.ShapeDtypeStruct((OUT_ROWS, 16), jnp.float32),
        scratch_types=[
            pltpu.VMEM((IDX_PER_CHUNK,), jnp.int32),
            pltpu.VMEM((IDX_PER_CHUNK,), jnp.int32),
            pltpu.VMEM((IDX_PER_CHUNK, EMBED), jnp.float32),
            pltpu.VMEM((IDX_PER_CHUNK, EMBED), jnp.float32),
            pltpu.VMEM((NCHUNK, 16), jnp.float32),
            pltpu.VMEM((5, 16), jnp.float32),
            pltpu.SemaphoreType.DMA,
            pltpu.SemaphoreType.DMA,
        ],
        compiler_params=pltpu.CompilerParams(use_tc_tiling_on_sc=False),
    )
    out16 = run(x2, table, params)
    return out16.reshape(BATCH, 2)
